# baseline (speedup 1.0000x reference)
import jax
import jax.numpy as jnp
from jax.experimental import pallas as pl

_RANGE_MIN = 24
_RANGE_MAX = 31
_IN_DIM = 128
_EMB_DIM = 18


def _body(x_ref, o_ref):
    v = x_ref[...]
    bb = v.shape[0]
    shape = (bb, _EMB_DIM, _IN_DIM)
    rep = jnp.broadcast_to(v[:, None, :], shape)
    c = jax.lax.broadcasted_iota(jnp.int32, shape, 2)
    e = jax.lax.broadcasted_iota(jnp.int32, shape, 1)
    is_card = (c >= _RANGE_MIN) & (c < _RANGE_MAX)
    vi = jnp.floor(rep)
    r = jnp.floor(vi * 0.25)
    s = vi - 4.0 * r
    ef = e.astype(jnp.float32)
    one = jnp.ones(shape, jnp.float32)
    zero = jnp.zeros(shape, jnp.float32)
    rank_oh = jnp.where(r == ef, one, zero)
    suit_oh = jnp.where(s == ef - 13.0, one, zero)
    card_val = jnp.where(e < 13, rank_oh, jnp.where(e < 17, suit_oh, one))
    o_ref[...] = jnp.where(is_card, card_val, rep)


@jax.jit
def _run(x2):
    b = x2.shape[0]
    bb = 2048
    out = pl.pallas_call(
        _body,
        grid=(b // bb,),
        in_specs=[pl.BlockSpec((bb, _IN_DIM), lambda i: (i, 0))],
        out_specs=pl.BlockSpec((bb, _EMB_DIM, _IN_DIM), lambda i: (i, 0, 0)),
        out_shape=jax.ShapeDtypeStruct((b, _EMB_DIM, _IN_DIM), jnp.float32),
    )(x2)
    return out.transpose(0, 2, 1)


def kernel(x):
    if x.ndim == 3:
        x = x[:, 0, :]
    return _run(x)


# speedup vs baseline: 1.0075x; 1.0075x over previous
import jax
import jax.numpy as jnp
from jax.experimental import pallas as pl

_RANGE_MIN = 24
_RANGE_MAX = 31
_IN_DIM = 128
_EMB_DIM = 18


def _body(x_ref, o_ref):
    v = x_ref[...]
    bb = v.shape[0]
    shape = (bb, _EMB_DIM, _IN_DIM)
    rep = jnp.broadcast_to(v[:, None, :], shape)
    c = jax.lax.broadcasted_iota(jnp.int32, shape, 2)
    e = jax.lax.broadcasted_iota(jnp.int32, shape, 1)
    is_card = (c >= _RANGE_MIN) & (c < _RANGE_MAX)
    vi = jnp.floor(rep)
    r = jnp.floor(vi * 0.25)
    s = vi - 4.0 * r
    ef = e.astype(jnp.float32)
    one = jnp.ones(shape, jnp.float32)
    zero = jnp.zeros(shape, jnp.float32)
    rank_oh = jnp.where(r == ef, one, zero)
    suit_oh = jnp.where(s == ef - 13.0, one, zero)
    card_val = jnp.where(e < 13, rank_oh, jnp.where(e < 17, suit_oh, one))
    o_ref[...] = jnp.where(is_card, card_val, rep)


@jax.jit
def _run(x2):
    b = x2.shape[0]
    bb = 1024
    out = pl.pallas_call(
        _body,
        grid=(b // bb,),
        in_specs=[pl.BlockSpec((bb, _IN_DIM), lambda i: (i, 0))],
        out_specs=pl.BlockSpec((bb, _EMB_DIM, _IN_DIM), lambda i: (i, 0, 0)),
        out_shape=jax.ShapeDtypeStruct((b, _EMB_DIM, _IN_DIM), jnp.float32),
    )(x2)
    return out.transpose(0, 2, 1)


def kernel(x):
    if x.ndim == 3:
        x = x[:, 0, :]
    return _run(x)


# speedup vs baseline: 4.3006x; 4.2684x over previous
import jax
import jax.numpy as jnp
from jax.experimental import pallas as pl

_RANGE_MIN = 24
_RANGE_MAX = 31
_IN_DIM = 128
_EMB_DIM = 18


def _body(x_ref, o_ref):
    v = x_ref[...]
    shape = (_EMB_DIM, v.shape[0], _IN_DIM)
    rep = jnp.broadcast_to(v[None, :, :], shape)
    c = jax.lax.broadcasted_iota(jnp.int32, shape, 2)
    e = jax.lax.broadcasted_iota(jnp.int32, shape, 0)
    is_card = (c >= _RANGE_MIN) & (c < _RANGE_MAX)
    vi = jnp.floor(rep)
    r = jnp.floor(vi * 0.25)
    s = vi - 4.0 * r
    ef = e.astype(jnp.float32)
    one = jnp.ones(shape, jnp.float32)
    zero = jnp.zeros(shape, jnp.float32)
    rank_oh = jnp.where(r == ef, one, zero)
    suit_oh = jnp.where(s == ef - 13.0, one, zero)
    card_val = jnp.where(e < 13, rank_oh, jnp.where(e < 17, suit_oh, one))
    o_ref[...] = jnp.where(is_card, card_val, rep)


@jax.jit
def _run(x2):
    b = x2.shape[0]
    bb = 1024
    out = pl.pallas_call(
        _body,
        grid=(b // bb,),
        in_specs=[pl.BlockSpec((bb, _IN_DIM), lambda i: (i, 0))],
        out_specs=pl.BlockSpec((_EMB_DIM, bb, _IN_DIM), lambda i: (0, i, 0)),
        out_shape=jax.ShapeDtypeStruct((_EMB_DIM, b, _IN_DIM), jnp.float32),
    )(x2)
    return out.transpose(1, 2, 0)


def kernel(x):
    if x.ndim == 3:
        x = x[:, 0, :]
    return _run(x)
